# Initial kernel scaffold; baseline (speedup 1.0000x reference)
import jax
import jax.numpy as jnp
from jax.experimental import pallas as pl


def kernel(logits, labels):
    raise NotImplementedError("write your pallas kernel here")


# speedup vs baseline: 1.7541x; 1.7541x over previous
import functools

import jax
import jax.numpy as jnp
from jax.experimental import pallas as pl
from jax.experimental.pallas import tpu as pltpu

_N = 16384
_NCM1 = 9
_NC = 10
_BLK = 2048


def _body(logits_ref, labels_ref, out_ref, acc_ref):
    step = pl.program_id(0)
    nsteps = pl.num_programs(0)

    @pl.when(step == 0)
    def _init():
        acc_ref[...] = jnp.zeros_like(acc_ref)

    sig = jax.nn.sigmoid(logits_ref[...])
    lab = labels_ref[...]
    col9 = jax.lax.broadcasted_iota(jnp.int32, (_BLK, _NCM1), 1)
    diffmask = (col9 == lab).astype(jnp.float32) - (col9 == lab + 1).astype(
        jnp.float32
    )
    gathered = jnp.sum(sig * diffmask, axis=1, keepdims=True) - (
        lab == _NCM1 - 1
    ).astype(jnp.float32)
    logt = jnp.log(gathered + 1e-9)

    col16 = jax.lax.broadcasted_iota(jnp.int32, (_BLK, 16), 1)
    oh = (col16 == lab).astype(jnp.float32)
    cnt = jnp.sum(oh, axis=0, keepdims=True)
    s = jnp.sum(oh * logt, axis=0, keepdims=True)
    acc_ref[0:1, :] += cnt
    acc_ref[1:2, :] += s

    @pl.when(step == nsteps - 1)
    def _finalize():
        counts = acc_ref[0:1, :]
        ssum = acc_ref[1:2, :]
        lane = jax.lax.broadcasted_iota(jnp.int32, (1, 16), 1)
        valid = lane < _NC
        total = jnp.sum(jnp.where(valid, counts, 0.0))
        w = counts / total
        w = jnp.where(w == 0.0, 1.0, w)
        inv = jnp.where(valid, 1.0 / w, 0.0)
        inv = inv / jnp.sum(inv)
        out_ref[...] = jnp.reshape(-jnp.sum(inv * ssum) / _N, (1, 1))


@jax.jit
def _loss(logits, labels2d):
    grid = _N // _BLK
    out = pl.pallas_call(
        _body,
        grid=(grid,),
        in_specs=[
            pl.BlockSpec((_BLK, _NCM1), lambda i: (i, 0)),
            pl.BlockSpec((_BLK, 1), lambda i: (i, 0)),
        ],
        out_specs=pl.BlockSpec((1, 1), lambda i: (0, 0)),
        out_shape=jax.ShapeDtypeStruct((1, 1), jnp.float32),
        scratch_shapes=[pltpu.VMEM((2, 16), jnp.float32)],
    )(logits, labels2d)
    return out[0, 0]


def kernel(logits, labels):
    logits = logits.reshape(-1, _NCM1)
    labels2d = labels.reshape(-1, 1).astype(jnp.int32)
    return _loss(logits, labels2d)


# speedup vs baseline: 14.2477x; 8.1227x over previous
import functools

import jax
import jax.numpy as jnp
from jax.experimental import pallas as pl
from jax.experimental.pallas import tpu as pltpu

_N = 16384
_NCM1 = 9
_NC = 10
_BLKL = 8192


def _body(logits_ref, labels_ref, out_ref, acc_ref):
    step = pl.program_id(0)
    nsteps = pl.num_programs(0)

    @pl.when(step == 0)
    def _init():
        acc_ref[...] = jnp.zeros_like(acc_ref)

    sig = jax.nn.sigmoid(logits_ref[...])
    lab = labels_ref[...]
    row9 = jax.lax.broadcasted_iota(jnp.int32, (_NCM1, _BLKL), 0)
    diffmask = (row9 == lab).astype(jnp.float32) - (row9 == lab + 1).astype(
        jnp.float32
    )
    gathered = jnp.sum(sig * diffmask, axis=0, keepdims=True) - (
        lab == _NCM1 - 1
    ).astype(jnp.float32)
    logt = jnp.log(gathered + 1e-9)

    row16 = jax.lax.broadcasted_iota(jnp.int32, (16, _BLKL), 0)
    oh = (row16 == lab).astype(jnp.float32)
    acc_ref[0:16, :] += oh
    acc_ref[16:32, :] += oh * logt

    @pl.when(step == nsteps - 1)
    def _finalize():
        counts = jnp.sum(acc_ref[0:16, :], axis=1, keepdims=True)
        ssum = jnp.sum(acc_ref[16:32, :], axis=1, keepdims=True)
        cls = jax.lax.broadcasted_iota(jnp.int32, (16, 1), 0)
        valid = cls < _NC
        total = jnp.sum(jnp.where(valid, counts, 0.0))
        w = counts / total
        w = jnp.where(w == 0.0, 1.0, w)
        inv = jnp.where(valid, 1.0 / w, 0.0)
        inv = inv / jnp.sum(inv)
        out_ref[...] = jnp.reshape(-jnp.sum(inv * ssum) / _N, (1, 1))


@jax.jit
def _loss(logits_t, labels2d):
    grid = _N // _BLKL
    out = pl.pallas_call(
        _body,
        grid=(grid,),
        in_specs=[
            pl.BlockSpec((_NCM1, _BLKL), lambda i: (0, i)),
            pl.BlockSpec((1, _BLKL), lambda i: (0, i)),
        ],
        out_specs=pl.BlockSpec((1, 1), lambda i: (0, 0)),
        out_shape=jax.ShapeDtypeStruct((1, 1), jnp.float32),
        scratch_shapes=[pltpu.VMEM((32, _BLKL), jnp.float32)],
    )(logits_t, labels2d)
    return out[0, 0]


def kernel(logits, labels):
    logits_t = logits.reshape(-1, _NCM1).T
    labels2d = labels.reshape(1, -1).astype(jnp.int32)
    return _loss(logits_t, labels2d)
